# baseline (speedup 1.0000x reference)
import jax
import jax.numpy as jnp
from jax.experimental import pallas as pl


def _stage1(x_ref, w_ref, s_ref):
    xt = x_ref[:, :].astype(jnp.bfloat16).T
    w = w_ref[:, :].astype(jnp.bfloat16)
    s = jnp.dot(xt, w, preferred_element_type=jnp.float32)
    s_ref[:, :] = s.astype(jnp.bfloat16)


def _stage2(adj_ref, s_ref, b_ref, out_ref):
    a = adj_ref[:, :].astype(jnp.bfloat16)
    acc = jnp.dot(a, s_ref[:, :], preferred_element_type=jnp.float32)
    acc = acc + b_ref[:, :]
    out_ref[:, :] = acc.T


def kernel(input, adj, weight, bias):
    C, N = input.shape
    F = weight.shape[1]

    s = pl.pallas_call(
        _stage1,
        in_specs=[
            pl.BlockSpec((C, N), lambda: (0, 0)),
            pl.BlockSpec((C, F), lambda: (0, 0)),
        ],
        out_specs=pl.BlockSpec((N, F), lambda: (0, 0)),
        out_shape=jax.ShapeDtypeStruct((N, F), jnp.bfloat16),
    )(input, weight)

    TN = 256
    bias2 = bias.reshape(1, F)
    out = pl.pallas_call(
        _stage2,
        grid=(pl.cdiv(N, TN),),
        in_specs=[
            pl.BlockSpec((TN, N), lambda i: (i, 0)),
            pl.BlockSpec((N, F), lambda i: (0, 0)),
            pl.BlockSpec((1, F), lambda i: (0, 0)),
        ],
        out_specs=pl.BlockSpec((F, TN), lambda i: (0, i)),
        out_shape=jax.ShapeDtypeStruct((F, N), jnp.float32),
    )(adj, s, bias2)
    return out


# speedup vs baseline: 1.0296x; 1.0296x over previous
import jax
import jax.numpy as jnp
from jax.experimental import pallas as pl
from jax.experimental.pallas import tpu as pltpu


def _fused(x_ref, w_ref, adj_ref, b_ref, out_ref, s_ref):
    i = pl.program_id(0)

    @pl.when(i == 0)
    def _():
        xt = x_ref[:, :].astype(jnp.bfloat16).T
        w = w_ref[:, :].astype(jnp.bfloat16)
        s = jnp.dot(xt, w, preferred_element_type=jnp.float32)
        s_ref[:, :] = s.astype(jnp.bfloat16)

    a = adj_ref[:, :].astype(jnp.bfloat16)
    acc = jnp.dot(a, s_ref[:, :], preferred_element_type=jnp.float32)
    acc = acc + b_ref[:, :]
    out_ref[:, :] = acc.T


def kernel(input, adj, weight, bias):
    C, N = input.shape
    F = weight.shape[1]

    TN = 256
    bias2 = bias.reshape(1, F)
    out = pl.pallas_call(
        _fused,
        grid=(pl.cdiv(N, TN),),
        in_specs=[
            pl.BlockSpec((C, N), lambda i: (0, 0)),
            pl.BlockSpec((C, F), lambda i: (0, 0)),
            pl.BlockSpec((TN, N), lambda i: (i, 0)),
            pl.BlockSpec((1, F), lambda i: (0, 0)),
        ],
        out_specs=pl.BlockSpec((F, TN), lambda i: (0, i)),
        out_shape=jax.ShapeDtypeStruct((F, N), jnp.float32),
        scratch_shapes=[pltpu.VMEM((N, F), jnp.bfloat16)],
    )(input, weight, adj, bias2)
    return out
